# Initial kernel scaffold; baseline (speedup 1.0000x reference)
import jax
import jax.numpy as jnp
from jax.experimental import pallas as pl


def kernel(x):
    raise NotImplementedError("write your pallas kernel here")


# speedup vs baseline: 11.1068x; 11.1068x over previous
import jax
import jax.numpy as jnp
from jax.experimental import pallas as pl


_BLOCK_ROWS = 4000


def _colsum_kernel(x_ref, o_ref):
    @pl.when(pl.program_id(0) == 0)
    def _init():
        o_ref[...] = jnp.zeros_like(o_ref)

    o_ref[...] += jnp.sum(x_ref[...], axis=0, keepdims=True)


def kernel(x):
    n, c = x.shape
    block = _BLOCK_ROWS
    grid = n // block
    out = pl.pallas_call(
        _colsum_kernel,
        grid=(grid,),
        in_specs=[pl.BlockSpec((block, c), lambda i: (i, 0))],
        out_specs=pl.BlockSpec((1, c), lambda i: (0, 0)),
        out_shape=jax.ShapeDtypeStruct((1, c), x.dtype),
    )(x)
    return out


# speedup vs baseline: 19.9875x; 1.7996x over previous
import jax
import jax.numpy as jnp
from jax.experimental import pallas as pl
from jax.experimental.pallas import tpu as pltpu


_BLOCK_ROWS = 16000


def _colsum_kernel(x_ref, o_ref, acc_ref):
    i = pl.program_id(0)

    @pl.when(i == 0)
    def _init():
        acc_ref[...] = jnp.zeros_like(acc_ref)

    acc_ref[...] += x_ref[...].reshape(-1, 32, 128).sum(axis=0)

    @pl.when(i == pl.num_programs(0) - 1)
    def _fini():
        o_ref[...] = acc_ref[...].sum(axis=0, keepdims=True)


def kernel(x):
    n, c = x.shape
    block = _BLOCK_ROWS
    grid = n // block
    out = pl.pallas_call(
        _colsum_kernel,
        grid=(grid,),
        in_specs=[pl.BlockSpec((block, c), lambda i: (i, 0))],
        out_specs=pl.BlockSpec((1, c), lambda i: (0, 0)),
        out_shape=jax.ShapeDtypeStruct((1, c), x.dtype),
        scratch_shapes=[pltpu.VMEM((32, c), jnp.float32)],
    )(x)
    return out
